# baseline (speedup 1.0000x reference)
import functools

import jax
import jax.numpy as jnp
from jax import lax
from jax.experimental import pallas as pl
from jax.experimental.pallas import tpu as pltpu
from jax.experimental.pallas import tpu_sc as plsc

N_NODES = 10000
D = 128
E = 320000
DELTA = 1.0

SEG = 10240
NC = 2
NS = 16
L = 16
NW = NC * NS
EPW = E // NW
C = 80
NCHUNK = EPW // C
GROUPS = C // L
NPT = SEG // NS
NEG_INIT = -3.0e38


def _shift(tmp_v, x, idx):
  tmp_v[pl.ds(0, L)] = x
  return plsc.load_gather(tmp_v, [idx])


def _seg_cnt_update(keys, ii, cnt_v, ktmp_v):
  ks, _ = plsc.sort_key_val(keys, keys)
  ktmp_v[pl.ds(0, L)] = ks
  k_prev = plsc.load_gather(ktmp_v, [jnp.maximum(ii - 1, 0)])
  is_head = (ii == 0) | (ks != k_prev)
  head = plsc.cummax(jnp.where(is_head, ii, 0))
  run_cnt = (ii - head + 1).astype(jnp.float32)
  k_next = plsc.load_gather(ktmp_v, [jnp.minimum(ii + 1, L - 1)])
  is_tail = (ii == L - 1) | (ks != k_next)
  old_c = plsc.load_gather(cnt_v, [ks])
  plsc.store_scatter(cnt_v, [ks], old_c + run_cnt, mask=is_tail)


def _seg_max_update(keys, dots, ii, max_v, flag_v, ktmp_v, vtmp_v):
  ks, vs = plsc.sort_key_val(keys, dots)
  ktmp_v[pl.ds(0, L)] = ks
  for d in (1, 2, 4, 8):
    idx = jnp.maximum(ii - d, 0)
    k_sh = plsc.load_gather(ktmp_v, [idx])
    cond = (ii >= d) & (ks == k_sh)
    vs = jnp.where(cond, jnp.maximum(vs, _shift(vtmp_v, vs, idx)), vs)
  k_next = plsc.load_gather(ktmp_v, [jnp.minimum(ii + 1, L - 1)])
  is_tail = (ii == L - 1) | (ks != k_next)
  old_m = plsc.load_gather(max_v, [ks])
  plsc.store_scatter(max_v, [ks], jnp.maximum(old_m, vs), mask=is_tail)
  plsc.store_scatter(flag_v, [ks], jnp.ones((L,), jnp.float32), mask=is_tail)


_MESH = plsc.VectorSubcoreMesh(core_axis_name="c", subcore_axis_name="s")


@functools.partial(
    pl.kernel,
    out_type=[jax.ShapeDtypeStruct((NW, SEG), jnp.float32)] * 4,
    mesh=_MESH,
    compiler_params=pltpu.CompilerParams(
        needs_layout_passes=False, use_tc_tiling_on_sc=False),
    scratch_types=[
        pltpu.VMEM((SEG,), jnp.int32),
        pltpu.VMEM((SEG,), jnp.float32),
        pltpu.VMEM((SEG,), jnp.float32),
        pltpu.VMEM((SEG,), jnp.float32),
        pltpu.VMEM((SEG,), jnp.float32),
        pltpu.VMEM((C,), jnp.int32),
        pltpu.VMEM((C,), jnp.int32),
        pltpu.VMEM((C,), jnp.int32),
        pltpu.VMEM((C,), jnp.int32),
        pltpu.VMEM((C,), jnp.int32),
        pltpu.VMEM((C,), jnp.int32),
        pltpu.VMEM((C,), jnp.int32),
        pltpu.VMEM((C,), jnp.int32),
        pltpu.VMEM((C,), jnp.int32),
        pltpu.VMEM((C, D // 2), jnp.float32),
        pltpu.VMEM((C, D // 2), jnp.float32),
        pltpu.VMEM((C, D // 2), jnp.int32),
        pltpu.VMEM((C, D // 2), jnp.int32),
        pltpu.VMEM((L,), jnp.int32),
        pltpu.VMEM((L,), jnp.float32),
        pltpu.VMEM_SHARED((SEG, D // 2), jnp.float32),
        pltpu.SemaphoreType.DMA,
        pltpu.SemaphoreType.DMA,
        pltpu.SemaphoreType.DMA,
        pltpu.SemaphoreType.DMA,
        pltpu.SemaphoreType.DMA,
        pltpu.SemaphoreType.DMA,
    ],
)
def _sc_partials(embpk_hbm, emblo_hbm, embhi_hbm, inv_hbm, psrc_hbm,
                 pdst_hbm, nsrc_hbm, ndst_hbm,
                 psum_out, pcnt_out, nmax_out, ncnt_out,
                 inv_v, psum_v, pcnt_v, nmax_v, ncnt_v,
                 src_p, dst_p, src_n, dst_n, keys_p, keys_n,
                 fdst_p, fsrc_n, fdst_n,
                 rows_pf, rows_ff, rows_na, rows_nb,
                 ktmp_v, vtmp_v, acc_sh,
                 sem_ep, sem_en, sem_rp, sem_rna, sem_rnb, sem_sp):
  sid = lax.axis_index("s")
  cid = lax.axis_index("c")
  wid = sid * NC + cid
  ii = lax.iota(jnp.int32, L)
  ebase = wid * EPW

  zero16 = jnp.zeros((L,), jnp.float32)
  neg16 = jnp.full((L,), NEG_INIT, jnp.float32)
  zero16i = jnp.zeros((L,), jnp.int32)

  def init_body(i, carry):
    off = i * L
    psum_v[pl.ds(off, L)] = zero16
    pcnt_v[pl.ds(off, L)] = zero16
    nmax_v[pl.ds(off, L)] = neg16
    ncnt_v[pl.ds(off, L)] = zero16
    return carry

  lax.fori_loop(0, SEG // L, init_body, 0)
  for k in range((SEG - N_NODES) // L):
    inv_v[pl.ds(N_NODES + k * L, L)] = zero16i

  pltpu.sync_copy(inv_hbm, inv_v.at[pl.ds(0, N_NODES)])

  def gather_half_rows(idx_ref, dst_ref, sem):
    @pl.when(cid == 0)
    def _():
      pltpu.async_copy(emblo_hbm.at[idx_ref], dst_ref, sem)

    @pl.when(cid == 1)
    def _():
      pltpu.async_copy(embhi_hbm.at[idx_ref], dst_ref, sem)

  def zrow_body(r, carry):
    for b in range(D // (2 * L)):
      rows_pf[r, pl.ds(b * L, L)] = zero16
    return carry

  lax.fori_loop(0, C, zrow_body, 0)
  for k in range(NPT // C):
    pltpu.sync_copy(rows_pf, acc_sh.at[pl.ds(sid * NPT + k * C, C)])
  plsc.subcore_barrier()

  def fire_edges(es_hbm, ed_hbm, ci, sv, dv, sem):
    base = ebase + ci * C
    pltpu.async_copy(es_hbm.at[pl.ds(base, C)], sv, sem)
    pltpu.async_copy(ed_hbm.at[pl.ds(base, C)], dv, sem)

  def wait_edges(es_hbm, ed_hbm, sv, dv, sem):
    pltpu.make_async_copy(es_hbm.at[pl.ds(0, C)], sv, sem).wait()
    pltpu.make_async_copy(ed_hbm.at[pl.ds(0, C)], dv, sem).wait()

  def compute_neg(rows_a, rows_b, kv):
    def group_body(g, gcarry):
      dots = jnp.zeros((L,), jnp.float32)
      for j in range(L):
        e = g * L + j
        acc = jnp.zeros((L,), jnp.float32)
        for b in range(D // (2 * L)):
          a_bf = plsc.bitcast(rows_a[e, pl.ds(b * L, L)], jnp.bfloat16)
          b_bf = plsc.bitcast(rows_b[e, pl.ds(b * L, L)], jnp.bfloat16)
          p_lo, p_hi = plsc.unpack(a_bf * b_bf,
                                   format=plsc.PackFormat.INTERLEAVED)
          acc = acc + p_lo + p_hi
        s = jnp.sum(acc)
        dots = jnp.where(ii == j, s, dots)
      keys = kv[pl.ds(g * L, L)]
      _seg_max_update(keys, dots, ii, nmax_v, ncnt_v, ktmp_v, vtmp_v)
      return gcarry

    lax.fori_loop(0, GROUPS, group_body, 0)

  fire_edges(psrc_hbm, pdst_hbm, 0, src_p, dst_p, sem_ep)

  def iter_body(i, carry):
    wait_edges(psrc_hbm, pdst_hbm, src_p, dst_p, sem_ep)

    @pl.when(i > 0)
    def _():
      pltpu.make_async_copy(rows_pf, acc_sh.at[keys_p], sem_sp).wait()

    for g in range(GROUPS):
      d16 = dst_p[pl.ds(g * L, L)]
      fdst_p[pl.ds(g * L, L)] = plsc.load_gather(inv_v, [d16])
      keys_p[pl.ds(g * L, L)] = src_p[pl.ds(g * L, L)]
    gather_half_rows(fdst_p, rows_pf, sem_rp)
    fire_edges(nsrc_hbm, ndst_hbm, i, src_n, dst_n, sem_en)

    @pl.when(i > 0)
    def _():
      pltpu.make_async_copy(embpk_hbm.at[fsrc_n], rows_na, sem_rna).wait()
      pltpu.make_async_copy(embpk_hbm.at[fdst_n], rows_nb, sem_rnb).wait()
      compute_neg(rows_na, rows_nb, keys_n)

    wait_edges(nsrc_hbm, ndst_hbm, src_n, dst_n, sem_en)
    for g in range(GROUPS):
      s16 = src_n[pl.ds(g * L, L)]
      d16 = dst_n[pl.ds(g * L, L)]
      fsrc_n[pl.ds(g * L, L)] = plsc.load_gather(inv_v, [s16])
      fdst_n[pl.ds(g * L, L)] = plsc.load_gather(inv_v, [d16])
      keys_n[pl.ds(g * L, L)] = s16
    pltpu.async_copy(embpk_hbm.at[fsrc_n], rows_na, sem_rna)
    pltpu.async_copy(embpk_hbm.at[fdst_n], rows_nb, sem_rnb)
    fire_edges(psrc_hbm, pdst_hbm, jnp.minimum(i + 1, NCHUNK - 1),
               src_p, dst_p, sem_ep)

    pltpu.make_async_copy(emblo_hbm.at[fdst_p], rows_pf, sem_rp).wait()
    pltpu.async_copy(rows_pf, acc_sh.at[keys_p], sem_sp, add=True)
    for g in range(GROUPS):
      _seg_cnt_update(keys_p[pl.ds(g * L, L)], ii, pcnt_v, ktmp_v)
    return carry

  lax.fori_loop(0, NCHUNK, iter_body, 0)

  pltpu.make_async_copy(rows_pf, acc_sh.at[keys_p], sem_sp).wait()
  pltpu.make_async_copy(embpk_hbm.at[fsrc_n], rows_na, sem_rna).wait()
  pltpu.make_async_copy(embpk_hbm.at[fdst_n], rows_nb, sem_rnb).wait()
  compute_neg(rows_na, rows_nb, keys_n)
  wait_edges(psrc_hbm, pdst_hbm, src_p, dst_p, sem_ep)

  plsc.subcore_barrier()

  def fin_body(k, carry):
    nbase = sid * NPT + k * C
    for g in range(GROUPS):
      node16 = nbase + g * L + ii
      fdst_p[pl.ds(g * L, L)] = plsc.load_gather(inv_v, [node16])
    gather_half_rows(fdst_p, rows_ff, sem_rp)
    pltpu.sync_copy(acc_sh.at[pl.ds(nbase, C)], rows_pf)
    pltpu.make_async_copy(emblo_hbm.at[fdst_p], rows_ff, sem_rp).wait()

    def fgroup(g, c2):
      dots = jnp.zeros((L,), jnp.float32)
      for j in range(L):
        e = g * L + j
        acc = rows_ff[e, pl.ds(0, L)] * rows_pf[e, pl.ds(0, L)]
        for b in range(1, D // (2 * L)):
          acc = acc + rows_ff[e, pl.ds(b * L, L)] * rows_pf[e, pl.ds(b * L, L)]
        s = jnp.sum(acc)
        dots = jnp.where(ii == j, s, dots)
      psum_v[pl.ds(nbase + g * L, L)] = dots
      return c2

    lax.fori_loop(0, GROUPS, fgroup, 0)
    return carry

  lax.fori_loop(0, NPT // C, fin_body, 0)

  pltpu.sync_copy(psum_v, psum_out.at[wid])
  pltpu.sync_copy(pcnt_v, pcnt_out.at[wid])
  pltpu.sync_copy(nmax_v, nmax_out.at[wid])
  pltpu.sync_copy(ncnt_v, ncnt_out.at[wid])


def _combine_body(psum_ref, pcnt_ref, nmax_ref, ncnt_ref, out_ref):
  p_sum = jnp.sum(psum_ref[...], axis=0, keepdims=True)
  p_cnt = jnp.sum(pcnt_ref[...], axis=0, keepdims=True)
  n_max = jnp.max(nmax_ref[...], axis=0, keepdims=True)
  n_cnt = jnp.sum(ncnt_ref[...], axis=0, keepdims=True)
  p_d = p_sum / jnp.maximum(p_cnt, 1.0)
  n_d = jnp.where(n_cnt > 0.0, n_max, 0.0)
  hinge = jnp.maximum(n_d - p_d + DELTA, 0.0)
  idx = lax.broadcasted_iota(jnp.int32, (1, SEG), 1)
  hinge = jnp.where(idx < N_NODES, hinge, 0.0)
  out_ref[...] = (jnp.sum(hinge) / N_NODES).reshape(1, 1)


def kernel(emb, inv_idx, pos_edges, neg_edges):
  pos_edges = pos_edges.astype(jnp.int32)
  neg_edges = neg_edges.astype(jnp.int32)
  emb_pk = lax.bitcast_convert_type(
      emb.astype(jnp.bfloat16).reshape(N_NODES, D // 2, 2), jnp.int32)
  emb2 = emb.reshape(N_NODES, 2, D // 2)
  psum, pcnt, nmax, ncnt = _sc_partials(
      emb_pk,
      emb2[:, 0, :],
      emb2[:, 1, :],
      inv_idx.astype(jnp.int32),
      pos_edges[0],
      pos_edges[1],
      neg_edges[0],
      neg_edges[1],
  )
  loss = pl.pallas_call(
      _combine_body,
      out_shape=jax.ShapeDtypeStruct((1, 1), jnp.float32),
  )(psum, pcnt, nmax, ncnt)
  return loss[0, 0]
